# baseline (speedup 1.0000x reference)
import jax
import jax.numpy as jnp
from jax.experimental import pallas as pl
from jax.experimental.pallas import tpu as pltpu

N = 2048
F = 256
H = 3
C = 32
BM = 256
NB = N // BM
FS0 = H * C
FN0 = H * C + 8


def _gat_kernel(x_ref, a_ref,
                We1_ref, b1_ref, We2_ref, b2_ref, We3_ref, b3_ref,
                Wf1_ref, bf1_ref, Wf2_ref, bf2_ref,
                out_ref,
                hx_s, fnT_s, rhs1_s, rhs2_s, aux_s, buf0, buf1, pmax):
    l = pl.program_id(0)
    b = pl.program_id(1)

    def compute_h(inp, We_ref):
        hx = jnp.dot(inp, We_ref[...], preferred_element_type=jnp.float32)
        hx_s[...] = hx
        aux_s[0:1, :] = jnp.mean(hx, axis=0, keepdims=True)
        fn_all = hx[:, FN0:FN0 + 8]
        fnmax = jnp.max(fn_all, axis=0, keepdims=True)
        aux_s[1:2, 0:8] = fnmax
        fnT_s[...] = fn_all.T
        d = fn_all - fnmax
        E1 = jnp.exp(d)
        E2 = jnp.exp(0.2 * d)
        for k in range(H):
            hk = hx[:, C * k:C * (k + 1)]
            rhs1_s[:, 128 * k:128 * k + C] = hk * E1[:, k:k + 1]
            rhs1_s[:, 128 * k + C:128 * k + C + 1] = E1[:, k:k + 1]
            rhs2_s[:, 128 * k:128 * k + C] = hk * E2[:, k:k + 1]
            rhs2_s[:, 128 * k + C:128 * k + C + 1] = E2[:, k:k + 1]

    @pl.when(jnp.logical_and(l == 0, b == 0))
    def _():
        compute_h(x_ref[...], We1_ref)

    @pl.when(jnp.logical_and(l == 1, b == 0))
    def _():
        compute_h(buf0[...], We2_ref)

    @pl.when(jnp.logical_and(l == 2, b == 0))
    def _():
        compute_h(buf1[...], We3_ref)

    M_f = a_ref[...].astype(jnp.float32)
    rows = pl.ds(b * BM, BM)
    acc = jnp.zeros((BM, C), jnp.float32)
    for k in range(H):
        fs_blk = hx_s[rows, FS0 + k:FS0 + k + 1]
        fnmax = aux_s[1:2, k:k + 1]
        t = fs_blk + fnmax
        m = jnp.maximum(t, 0.2 * t)
        a1 = jnp.exp(t - m)
        a2 = jnp.exp(0.2 * t - m)
        s = fnT_s[k:k + 1, :] >= -fs_blk
        P1 = jnp.where(s, M_f, 0.0)
        P2 = M_f - P1
        Q1 = jnp.dot(P1, rhs1_s[:, 128 * k:128 * k + C + 1],
                     preferred_element_type=jnp.float32)
        Q2 = jnp.dot(P2, rhs2_s[:, 128 * k:128 * k + C + 1],
                     preferred_element_type=jnp.float32)
        num = a1 * Q1[:, :C] + a2 * Q2[:, :C]
        den = a1 * Q1[:, C:C + 1] + a2 * Q2[:, C:C + 1]
        r = jnp.where(den > 0, 1.0 / den, 0.0)
        acc = acc + jnp.where(den > 0, num * r,
                              aux_s[0:1, C * k:C * (k + 1)])

    @pl.when(l == 0)
    def _():
        buf0[rows, :] = jnp.maximum(acc * (1.0 / H) + b1_ref[...], 0.0)

    @pl.when(l == 1)
    def _():
        buf1[rows, :] = jnp.maximum(acc * (1.0 / H) + b2_ref[...], 0.0)

    @pl.when(l == 2)
    def _():
        xo = jnp.maximum(acc * (1.0 / H) + b3_ref[...], 0.0)
        bmax = jnp.max(xo, axis=0, keepdims=True)
        prev = jnp.where(b == 0, -jnp.inf, pmax[...])
        pmax[...] = jnp.maximum(prev, bmax)

    @pl.when(jnp.logical_and(l == 2, b == NB - 1))
    def _():
        p = pmax[...]
        hf = jnp.maximum(
            jnp.dot(p, Wf1_ref[...], preferred_element_type=jnp.float32)
            + bf1_ref[...], 0.0)
        out_ref[...] = (jnp.dot(hf, Wf2_ref[...],
                                preferred_element_type=jnp.float32)
                        + bf2_ref[...])


def _fold(W, a_s, a_n):
    f = W.shape[0]
    sel = jnp.repeat(jnp.arange(H), C)
    bd_s = jnp.where(sel[:, None] == jnp.arange(H)[None, :],
                     a_s.reshape(-1)[:, None], 0.0)
    bd_n = jnp.where(sel[:, None] == jnp.arange(H)[None, :],
                     a_n.reshape(-1)[:, None], 0.0)
    pad5 = jnp.zeros((f, 5), jnp.float32)
    return jnp.concatenate(
        [W, W @ bd_s, pad5, W @ bd_n, pad5,
         jnp.zeros((f, 128 - H * C - 16), jnp.float32)], axis=1)


def kernel(x, W1, as1, an1, b1, W2, as2, an2, b2, W3, as3, an3, b3,
           Wf1, bf1, Wf2, bf2, a):
    a8 = a.astype(jnp.int8)
    We1 = _fold(W1, as1, an1)
    We2 = _fold(W2, as2, an2)
    We3 = _fold(W3, as3, an3)

    def const(shape):
        return pl.BlockSpec(shape, lambda l, b: (0,) * len(shape))

    in_specs = [
        pl.BlockSpec((N, F), lambda l, b: (0, 0)),
        pl.BlockSpec((BM, N), lambda l, b: (b, 0)),
        const((F, 128)), const((1, C)),
        const((C, 128)), const((1, C)),
        const((C, 128)), const((1, C)),
        const((C, 2 * C)), const((1, 2 * C)),
        const((2 * C, 1)), const((1, 1)),
    ]
    out = pl.pallas_call(
        _gat_kernel,
        grid=(3, NB),
        in_specs=in_specs,
        out_specs=pl.BlockSpec((1, 1), lambda l, b: (0, 0)),
        out_shape=jax.ShapeDtypeStruct((1, 1), jnp.float32),
        scratch_shapes=[
            pltpu.VMEM((N, 128), jnp.float32),
            pltpu.VMEM((8, N), jnp.float32),
            pltpu.VMEM((N, 128 * H), jnp.float32),
            pltpu.VMEM((N, 128 * H), jnp.float32),
            pltpu.VMEM((8, 128), jnp.float32),
            pltpu.VMEM((N, C), jnp.float32),
            pltpu.VMEM((N, C), jnp.float32),
            pltpu.VMEM((1, C), jnp.float32),
        ],
        compiler_params=pltpu.CompilerParams(
            dimension_semantics=("arbitrary", "arbitrary")),
    )(x, a8, We1, b1.reshape(1, C), We2, b2.reshape(1, C),
      We3, b3.reshape(1, C), Wf1, bf1.reshape(1, 2 * C),
      Wf2, bf2.reshape(1, 1))
    return out
